# baseline (speedup 1.0000x reference)
import functools

import jax
import jax.numpy as jnp
from jax import lax
from jax.experimental import pallas as pl
from jax.experimental.pallas import tpu as pltpu
from jax.experimental.pallas import tpu_sc as plsc

N = 10000
E = 160000
D = 256
H = 128
NC, NS = 2, 16
CHUNK = 128
EPT = 80
EPAD = NS * EPT * CHUNK
SB = 8
NSB = EPT // SB
NSLAB = 10112
RPT = NSLAB // NS
LAST = N - 15 * RPT
CPT = NSLAB // NS

_sc_mesh = plsc.VectorSubcoreMesh(core_axis_name="c", subcore_axis_name="s")


def _sc_segsum_build(with_cnt):
    out_type = [
        jax.ShapeDtypeStruct((N, H), jnp.float32),
        jax.ShapeDtypeStruct((N, H), jnp.float32),
    ]
    if with_cnt:
        out_type.append(jax.ShapeDtypeStruct((NSLAB,), jnp.float32))

    @functools.partial(
        pl.kernel,
        mesh=_sc_mesh,
        out_type=out_type,
        scratch_types=[
            pltpu.VMEM_SHARED((NSLAB, H), jnp.float32),
            pltpu.VMEM_SHARED((NSLAB,), jnp.float32),
            pltpu.VMEM((2, SB, CHUNK), jnp.int32),
            pltpu.VMEM((2, SB, CHUNK), jnp.int32),
            pltpu.VMEM((2, CHUNK, H), jnp.float32),
            pltpu.VMEM((CHUNK,), jnp.float32),
            pltpu.VMEM((640,), jnp.float32),
            pltpu.SemaphoreType.DMA,
            pltpu.SemaphoreType.DMA,
            pltpu.SemaphoreType.DMA,
        ],
        compiler_params=pltpu.CompilerParams(use_tc_tiling_on_sc=True),
    )
    def _sc_segsum(x0_hbm, x1_hbm, src_hbm, dst_hbm, zer_hbm, *rest):
        if with_cnt:
            (out0, out1, cnt_out, agg_sh, cnt_sh, sidx, didx, rows, ones,
             cbuf, gsem0, gsem1, isem) = rest
        else:
            (out0, out1, agg_sh, cnt_sh, sidx, didx, rows, ones,
             cbuf, gsem0, gsem1, isem) = rest
        c = lax.axis_index("c")
        s = lax.axis_index("s")
        gsems = (gsem0, gsem1)
        @pl.when(s < NS - 1)
        def _z():
            pltpu.sync_copy(zer_hbm.at[pl.ds(s * RPT, RPT)],
                            agg_sh.at[pl.ds(s * RPT, RPT)])
        @pl.when(s == NS - 1)
        def _zl():
            pltpu.sync_copy(zer_hbm.at[pl.ds(15 * RPT, LAST)],
                            agg_sh.at[pl.ds(15 * RPT, LAST)])
        if with_cnt:
            for i in range(640 // 16):
                cbuf[pl.ds(i * 16, 16)] = jnp.zeros((16,), jnp.float32)
            @pl.when(c == 0)
            def _zc():
                pltpu.sync_copy(cbuf.at[pl.ds(0, CPT)],
                                cnt_sh.at[pl.ds(s * CPT, CPT)])
            for i in range(CHUNK // 16):
                ones[pl.ds(i * 16, 16)] = jnp.ones((16,), jnp.float32)
        plsc.subcore_barrier()

        base = s * EPT

        def _idx_load(S, ib):
            pltpu.async_copy(src_hbm.at[pl.ds(base + S * SB, SB)],
                             sidx.at[ib], isem)
            pltpu.async_copy(dst_hbm.at[pl.ds(base + S * SB, SB)],
                             didx.at[ib], isem)

        def _idx_wait(S, ib):
            pltpu.make_async_copy(src_hbm.at[pl.ds(base + S * SB, SB)],
                                  sidx.at[ib], isem).wait()
            pltpu.make_async_copy(dst_hbm.at[pl.ds(base + S * SB, SB)],
                                  didx.at[ib], isem).wait()

        def _gather_start(ib, j, b):
            @pl.when(c == 0)
            def _g0():
                pltpu.async_copy(x0_hbm.at[sidx.at[ib, j]], rows.at[b],
                                 gsems[b])
            @pl.when(c == 1)
            def _g1():
                pltpu.async_copy(x1_hbm.at[sidx.at[ib, j]], rows.at[b],
                                 gsems[b])

        def _gather_wait(ib, j, b):
            pltpu.make_async_copy(x0_hbm.at[sidx.at[ib, j]], rows.at[b],
                                  gsems[b]).wait()

        _idx_load(0, 0)
        _idx_wait(0, 0)
        _gather_start(0, 0, 0)
        _idx_load(1, 1)

        @pl.loop(0, NSB, step=2)
        def _pipeline(Sb):
            for sb in range(2):
                S = Sb + sb
                ib = sb
                for j in range(SB):
                    b = j % 2
                    if j < SB - 1:
                        _gather_start(ib, j + 1, 1 - b)
                    else:
                        @pl.when(S + 1 < NSB)
                        def _pf():
                            _idx_wait(S + 1, 1 - ib)
                            _gather_start(1 - ib, 0, 1 - b)
                    _gather_wait(ib, j, b)
                    pltpu.sync_copy(rows.at[b], agg_sh.at[didx.at[ib, j]],
                                    add=True)
                    if with_cnt:
                        @pl.when(c == 0)
                        def _cnt():
                            pltpu.sync_copy(ones,
                                            cnt_sh.at[didx.at[ib, j]],
                                            add=True)
                    if j == SB - 1:
                        @pl.when(S + 2 < NSB)
                        def _pf2():
                            _idx_load(S + 2, ib)

        plsc.subcore_barrier()
        def _writeout(out):
            @pl.when(s < NS - 1)
            def _w():
                pltpu.sync_copy(agg_sh.at[pl.ds(s * RPT, RPT)],
                                out.at[pl.ds(s * RPT, RPT)])
            @pl.when(s == NS - 1)
            def _wl():
                pltpu.sync_copy(agg_sh.at[pl.ds(15 * RPT, LAST)],
                                out.at[pl.ds(15 * RPT, LAST)])
        @pl.when(c == 0)
        def _w0():
            _writeout(out0)
            if with_cnt:
                pltpu.sync_copy(cnt_sh.at[pl.ds(s * CPT, CPT)],
                                cbuf.at[pl.ds(0, CPT)])
                pltpu.sync_copy(cbuf.at[pl.ds(0, CPT)],
                                cnt_out.at[pl.ds(s * CPT, CPT)])
        @pl.when(c == 1)
        def _w1():
            _writeout(out1)

    return _sc_segsum


_sc_segsum_cnt = _sc_segsum_build(True)
_sc_segsum_nocnt = _sc_segsum_build(False)


BM = 1000


def _tc_compute(a0_ref, a1_ref, x0_ref, x1_ref, cnt_ref, wl_ref, bl_ref,
                wr_ref):
    aggf = jnp.concatenate([a0_ref[...], a1_ref[...]], axis=1)
    xf = jnp.concatenate([x0_ref[...], x1_ref[...]], axis=1)
    mean = aggf / jnp.maximum(cnt_ref[...], 1.0)
    return (jnp.dot(mean, wl_ref[...], preferred_element_type=jnp.float32)
            + jnp.dot(xf, wr_ref[...], preferred_element_type=jnp.float32)
            + bl_ref[...])


def _tc_body_split(a0_ref, a1_ref, x0_ref, x1_ref, cnt_ref, wl_ref, bl_ref,
                   wr_ref, o0_ref, o1_ref):
    res = _tc_compute(a0_ref, a1_ref, x0_ref, x1_ref, cnt_ref, wl_ref,
                      bl_ref, wr_ref)
    o0_ref[...] = res[:, :H]
    o1_ref[...] = res[:, H:]


def _tc_body_final(a0_ref, a1_ref, x0_ref, x1_ref, cnt_ref, wl_ref, bl_ref,
                   wr_ref, o_ref):
    o_ref[...] = _tc_compute(a0_ref, a1_ref, x0_ref, x1_ref, cnt_ref,
                             wl_ref, bl_ref, wr_ref)


def _tc_layer(a0, a1, x0, x1, cnt, Wl, bl, Wr, final):
    half = pl.BlockSpec((BM, H), lambda i: (i, 0))
    in_specs = [
        half, half, half, half,
        pl.BlockSpec((BM, 1), lambda i: (i, 0)),
        pl.BlockSpec((D, D), lambda i: (0, 0)),
        pl.BlockSpec((1, D), lambda i: (0, 0)),
        pl.BlockSpec((D, D), lambda i: (0, 0)),
    ]
    if final:
        out_shape = jax.ShapeDtypeStruct((N, D), jnp.float32)
        out_spec = pl.BlockSpec((BM, D), lambda i: (i, 0))
        body = _tc_body_final
    else:
        out_shape = [jax.ShapeDtypeStruct((N, H), jnp.float32)] * 2
        out_spec = [half, half]
        body = _tc_body_split
    return pl.pallas_call(
        body,
        grid=(N // BM,),
        in_specs=in_specs,
        out_specs=out_spec,
        out_shape=out_shape,
    )(a0, a1, x0, x1, cnt, Wl, bl.reshape(1, D), Wr)


def kernel(x, edge_index, Wl0, bl0, Wr0, Wl1, bl1, Wr1):
    src = edge_index[0].astype(jnp.int32)
    dst = edge_index[1].astype(jnp.int32)
    npad = EPAD - E
    srcr = jnp.concatenate(
        [src, jnp.zeros((npad,), jnp.int32)]).reshape(EPAD // CHUNK, CHUNK)
    dstr = jnp.concatenate(
        [dst, jnp.full((npad,), N, jnp.int32)]).reshape(EPAD // CHUNK, CHUNK)
    zer = jnp.zeros((N, H), jnp.float32)
    x0 = x[:, :H]
    x1 = x[:, H:]

    a0, a1, cnt = _sc_segsum_cnt(x0, x1, srcr, dstr, zer)
    cnt2 = cnt[:N].reshape(N, 1)
    h0, h1 = _tc_layer(a0, a1, x0, x1, cnt2, Wl0, bl0, Wr0, final=False)
    b0, b1 = _sc_segsum_nocnt(h0, h1, srcr, dstr, zer)
    return _tc_layer(b0, b1, h0, h1, cnt2, Wl1, bl1, Wr1, final=True)


# speedup vs baseline: 2.4706x; 2.4706x over previous
import functools

import jax
import jax.numpy as jnp
from jax import lax
from jax.experimental import pallas as pl
from jax.experimental.pallas import tpu as pltpu
from jax.experimental.pallas import tpu_sc as plsc

N = 10000
E = 160000
D = 256
H = 128
NC, NS = 2, 16
CHUNK = 125
EPT = E // (NS * CHUNK)
SB = 8
NSB = EPT // SB
RPT = N // NS
NCNT = 10240
CPT = NCNT // NS

_sc_mesh = plsc.VectorSubcoreMesh(core_axis_name="c", subcore_axis_name="s")


def _sc_segsum_build(with_cnt):
    out_type = [
        jax.ShapeDtypeStruct((N, H), jnp.float32),
        jax.ShapeDtypeStruct((N, H), jnp.float32),
    ]
    if with_cnt:
        out_type.append(jax.ShapeDtypeStruct((NCNT,), jnp.float32))

    @functools.partial(
        pl.kernel,
        mesh=_sc_mesh,
        out_type=out_type,
        scratch_types=[
            pltpu.VMEM_SHARED((N, H), jnp.float32),
            pltpu.VMEM_SHARED((NCNT,), jnp.float32),
            pltpu.VMEM((2, SB, CHUNK), jnp.int32),
            pltpu.VMEM((2, SB, CHUNK), jnp.int32),
            pltpu.VMEM((2, CHUNK, H), jnp.float32),
            pltpu.VMEM((128,), jnp.float32),
            pltpu.SemaphoreType.DMA,
            pltpu.SemaphoreType.DMA,
            pltpu.SemaphoreType.DMA,
        ],
        compiler_params=pltpu.CompilerParams(use_tc_tiling_on_sc=False),
    )
    def _sc_segsum(x0_hbm, x1_hbm, ei_hbm, *rest):
        if with_cnt:
            (out0, out1, cnt_out, agg_sh, cnt_sh, sidx, didx, rows, ones,
             gsem0, gsem1, isem) = rest
        else:
            (out0, out1, agg_sh, cnt_sh, sidx, didx, rows, ones,
             gsem0, gsem1, isem) = rest
        c = lax.axis_index("c")
        s = lax.axis_index("s")
        gsems = (gsem0, gsem1)
        @pl.loop(0, CHUNK)
        def _zr(r):
            for i in range(H // 16):
                rows[0, r, pl.ds(i * 16, 16)] = jnp.zeros((16,), jnp.float32)
        for q in range(RPT // CHUNK):
            pltpu.sync_copy(rows.at[0],
                            agg_sh.at[pl.ds(s * RPT + q * CHUNK, CHUNK)])
        if with_cnt:
            for q in range(CPT // H):
                pltpu.sync_copy(rows.at[0, 0],
                                cnt_sh.at[pl.ds(s * CPT + q * H, H)])
            for i in range(8):
                ones[pl.ds(i * 16, 16)] = jnp.ones((16,), jnp.float32)
        plsc.subcore_barrier()

        base = s * EPT

        def _idx_load(S, ib):
            pltpu.async_copy(ei_hbm.at[0, pl.ds(base + S * SB, SB)],
                             sidx.at[ib], isem)
            pltpu.async_copy(ei_hbm.at[1, pl.ds(base + S * SB, SB)],
                             didx.at[ib], isem)

        def _idx_wait(S, ib):
            pltpu.make_async_copy(ei_hbm.at[0, pl.ds(base + S * SB, SB)],
                                  sidx.at[ib], isem).wait()
            pltpu.make_async_copy(ei_hbm.at[1, pl.ds(base + S * SB, SB)],
                                  didx.at[ib], isem).wait()

        def _gather_start(ib, j, b):
            @pl.when(c == 0)
            def _g0():
                pltpu.async_copy(x0_hbm.at[sidx.at[ib, j]], rows.at[b],
                                 gsems[b])
            @pl.when(c == 1)
            def _g1():
                pltpu.async_copy(x1_hbm.at[sidx.at[ib, j]], rows.at[b],
                                 gsems[b])

        def _gather_wait(ib, j, b):
            pltpu.make_async_copy(x0_hbm.at[sidx.at[ib, j]], rows.at[b],
                                  gsems[b]).wait()

        _idx_load(0, 0)
        _idx_wait(0, 0)
        _gather_start(0, 0, 0)
        _idx_load(1, 1)

        @pl.loop(0, NSB, step=2)
        def _pipeline(Sb):
            for sb in range(2):
                S = Sb + sb
                ib = sb
                for j in range(SB):
                    b = j % 2
                    if j < SB - 1:
                        _gather_start(ib, j + 1, 1 - b)
                    else:
                        @pl.when(S + 1 < NSB)
                        def _pf():
                            _idx_wait(S + 1, 1 - ib)
                            _gather_start(1 - ib, 0, 1 - b)
                    _gather_wait(ib, j, b)
                    pltpu.sync_copy(rows.at[b], agg_sh.at[didx.at[ib, j]],
                                    add=True)
                    if with_cnt:
                        @pl.when(c == 0)
                        def _cnt():
                            pltpu.sync_copy(ones.at[pl.ds(0, CHUNK)],
                                            cnt_sh.at[didx.at[ib, j]],
                                            add=True)
                    if j == SB - 1:
                        @pl.when(S + 2 < NSB)
                        def _pf2():
                            _idx_load(S + 2, ib)

        plsc.subcore_barrier()
        @pl.when(c == 0)
        def _w0():
            pltpu.sync_copy(agg_sh.at[pl.ds(s * RPT, RPT)],
                            out0.at[pl.ds(s * RPT, RPT)])
            if with_cnt:
                pltpu.sync_copy(cnt_sh.at[pl.ds(s * CPT, CPT)],
                                cnt_out.at[pl.ds(s * CPT, CPT)])
        @pl.when(c == 1)
        def _w1():
            pltpu.sync_copy(agg_sh.at[pl.ds(s * RPT, RPT)],
                            out1.at[pl.ds(s * RPT, RPT)])

    return _sc_segsum


_sc_segsum_cnt = _sc_segsum_build(True)
_sc_segsum_nocnt = _sc_segsum_build(False)


BM = 1000


def _tc_compute(a0_ref, a1_ref, x0_ref, x1_ref, cnt_ref, wl_ref, bl_ref,
                wr_ref):
    aggf = jnp.concatenate([a0_ref[...], a1_ref[...]], axis=1)
    xf = jnp.concatenate([x0_ref[...], x1_ref[...]], axis=1)
    mean = aggf / jnp.maximum(cnt_ref[...], 1.0)
    return (jnp.dot(mean, wl_ref[...], preferred_element_type=jnp.float32)
            + jnp.dot(xf, wr_ref[...], preferred_element_type=jnp.float32)
            + bl_ref[...])


def _tc_body_split(a0_ref, a1_ref, x0_ref, x1_ref, cnt_ref, wl_ref, bl_ref,
                   wr_ref, o0_ref, o1_ref):
    res = _tc_compute(a0_ref, a1_ref, x0_ref, x1_ref, cnt_ref, wl_ref,
                      bl_ref, wr_ref)
    o0_ref[...] = res[:, :H]
    o1_ref[...] = res[:, H:]


def _tc_body_final(a0_ref, a1_ref, x0_ref, x1_ref, cnt_ref, wl_ref, bl_ref,
                   wr_ref, o_ref):
    o_ref[...] = _tc_compute(a0_ref, a1_ref, x0_ref, x1_ref, cnt_ref,
                             wl_ref, bl_ref, wr_ref)


def _tc_layer(a0, a1, x0, x1, cnt, Wl, bl, Wr, final):
    half = pl.BlockSpec((BM, H), lambda i: (i, 0))
    in_specs = [
        half, half, half, half,
        pl.BlockSpec((BM, 1), lambda i: (i, 0)),
        pl.BlockSpec((D, D), lambda i: (0, 0)),
        pl.BlockSpec((1, D), lambda i: (0, 0)),
        pl.BlockSpec((D, D), lambda i: (0, 0)),
    ]
    if final:
        out_shape = jax.ShapeDtypeStruct((N, D), jnp.float32)
        out_spec = pl.BlockSpec((BM, D), lambda i: (i, 0))
        body = _tc_body_final
    else:
        out_shape = [jax.ShapeDtypeStruct((N, H), jnp.float32)] * 2
        out_spec = [half, half]
        body = _tc_body_split
    return pl.pallas_call(
        body,
        grid=(N // BM,),
        in_specs=in_specs,
        out_specs=out_spec,
        out_shape=out_shape,
    )(a0, a1, x0, x1, cnt, Wl, bl.reshape(1, D), Wr)


def kernel(x, edge_index, Wl0, bl0, Wr0, Wl1, bl1, Wr1):
    ei3 = edge_index.astype(jnp.int32).reshape(2, E // CHUNK, CHUNK)
    x0 = x[:, :H]
    x1 = x[:, H:]

    a0, a1, cnt = _sc_segsum_cnt(x0, x1, ei3)
    cnt2 = cnt[:N].reshape(N, 1)
    h0, h1 = _tc_layer(a0, a1, x0, x1, cnt2, Wl0, bl0, Wr0, final=False)
    b0, b1 = _sc_segsum_nocnt(h0, h1, ei3)
    return _tc_layer(b0, b1, h0, h1, cnt2, Wl1, bl1, Wr1, final=True)


# speedup vs baseline: 2.4987x; 1.0114x over previous
import functools

import jax
import jax.numpy as jnp
from jax import lax
from jax.experimental import pallas as pl
from jax.experimental.pallas import tpu as pltpu
from jax.experimental.pallas import tpu_sc as plsc

N = 10000
E = 160000
D = 256
H = 128
NC, NS = 2, 16
CHUNK = 125
EPT = E // (NS * CHUNK)
SB = 8
NSB = EPT // SB
RPT = N // NS
NCNT = 10240
CPT = NCNT // NS

_sc_mesh = plsc.VectorSubcoreMesh(core_axis_name="c", subcore_axis_name="s")


def _sc_segsum_build(with_cnt):
    out_type = [
        jax.ShapeDtypeStruct((N, H), jnp.float32),
        jax.ShapeDtypeStruct((N, H), jnp.float32),
    ]
    if with_cnt:
        out_type.append(jax.ShapeDtypeStruct((NCNT,), jnp.float32))

    @functools.partial(
        pl.kernel,
        mesh=_sc_mesh,
        out_type=out_type,
        scratch_types=[
            pltpu.VMEM_SHARED((N, H), jnp.float32),
            pltpu.VMEM_SHARED((NCNT,), jnp.float32),
            pltpu.VMEM((2, SB, CHUNK), jnp.int32),
            pltpu.VMEM((2, SB, CHUNK), jnp.int32),
            pltpu.VMEM((2, CHUNK, H), jnp.float32),
            pltpu.VMEM((128,), jnp.float32),
            pltpu.SemaphoreType.DMA,
            pltpu.SemaphoreType.DMA,
            pltpu.SemaphoreType.DMA,
        ],
        compiler_params=pltpu.CompilerParams(use_tc_tiling_on_sc=False),
    )
    def _sc_segsum(x0_hbm, x1_hbm, ei_hbm, *rest):
        if with_cnt:
            (out0, out1, cnt_out, agg_sh, cnt_sh, sidx, didx, rows, ones,
             gsem0, gsem1, isem) = rest
        else:
            (out0, out1, agg_sh, cnt_sh, sidx, didx, rows, ones,
             gsem0, gsem1, isem) = rest
        c = lax.axis_index("c")
        s = lax.axis_index("s")
        gsems = (gsem0, gsem1)
        @pl.loop(0, CHUNK)
        def _zr(r):
            for i in range(H // 16):
                rows[0, r, pl.ds(i * 16, 16)] = jnp.zeros((16,), jnp.float32)
        for q in range(RPT // CHUNK):
            pltpu.sync_copy(rows.at[0],
                            agg_sh.at[pl.ds(s * RPT + q * CHUNK, CHUNK)])
        if with_cnt:
            for q in range(CPT // H):
                pltpu.sync_copy(rows.at[0, 0],
                                cnt_sh.at[pl.ds(s * CPT + q * H, H)])
            for i in range(8):
                ones[pl.ds(i * 16, 16)] = jnp.ones((16,), jnp.float32)
        plsc.subcore_barrier()

        base = s * EPT

        def _idx_load(S, ib):
            pltpu.async_copy(ei_hbm.at[0, pl.ds(base + S * SB, SB)],
                             sidx.at[ib], isem)
            pltpu.async_copy(ei_hbm.at[1, pl.ds(base + S * SB, SB)],
                             didx.at[ib], isem)

        def _idx_wait(S, ib):
            pltpu.make_async_copy(ei_hbm.at[0, pl.ds(base + S * SB, SB)],
                                  sidx.at[ib], isem).wait()
            pltpu.make_async_copy(ei_hbm.at[1, pl.ds(base + S * SB, SB)],
                                  didx.at[ib], isem).wait()

        def _gather_start(ib, j, b):
            @pl.when(c == 0)
            def _g0():
                pltpu.async_copy(x0_hbm.at[sidx.at[ib, j]], rows.at[b],
                                 gsems[b])
            @pl.when(c == 1)
            def _g1():
                pltpu.async_copy(x1_hbm.at[sidx.at[ib, j]], rows.at[b],
                                 gsems[b])

        def _gather_wait(ib, j, b):
            pltpu.make_async_copy(x0_hbm.at[sidx.at[ib, j]], rows.at[b],
                                  gsems[b]).wait()

        _idx_load(0, 0)
        _idx_wait(0, 0)
        _gather_start(0, 0, 0)
        _idx_load(1, 1)

        @pl.loop(0, NSB, step=2)
        def _pipeline(Sb):
            for sb in range(2):
                S = Sb + sb
                ib = sb
                for j in range(SB):
                    b = j % 2
                    if j < SB - 1:
                        _gather_start(ib, j + 1, 1 - b)
                    else:
                        @pl.when(S + 1 < NSB)
                        def _pf():
                            _idx_wait(S + 1, 1 - ib)
                            _gather_start(1 - ib, 0, 1 - b)
                    _gather_wait(ib, j, b)
                    pltpu.sync_copy(rows.at[b], agg_sh.at[didx.at[ib, j]],
                                    add=True)
                    if with_cnt:
                        @pl.when(c == 0)
                        def _cnt():
                            pltpu.sync_copy(ones.at[pl.ds(0, CHUNK)],
                                            cnt_sh.at[didx.at[ib, j]],
                                            add=True)
                    if j == SB - 1:
                        @pl.when(S + 2 < NSB)
                        def _pf2():
                            _idx_load(S + 2, ib)

        plsc.subcore_barrier()
        @pl.when(c == 0)
        def _w0():
            pltpu.sync_copy(agg_sh.at[pl.ds(s * RPT, RPT)],
                            out0.at[pl.ds(s * RPT, RPT)])
            if with_cnt:
                pltpu.sync_copy(cnt_sh.at[pl.ds(s * CPT, CPT)],
                                cnt_out.at[pl.ds(s * CPT, CPT)])
        @pl.when(c == 1)
        def _w1():
            pltpu.sync_copy(agg_sh.at[pl.ds(s * RPT, RPT)],
                            out1.at[pl.ds(s * RPT, RPT)])

    return _sc_segsum


_sc_segsum_cnt = _sc_segsum_build(True)
_sc_segsum_nocnt = _sc_segsum_build(False)


BM = 2000


def _tc_compute(a0_ref, a1_ref, x0_ref, x1_ref, cnt_ref, wl_ref, bl_ref,
                wr_ref):
    aggf = jnp.concatenate([a0_ref[...], a1_ref[...]], axis=1)
    xf = jnp.concatenate([x0_ref[...], x1_ref[...]], axis=1)
    mean = aggf / jnp.maximum(cnt_ref[...], 1.0)
    return (jnp.dot(mean, wl_ref[...], preferred_element_type=jnp.float32)
            + jnp.dot(xf, wr_ref[...], preferred_element_type=jnp.float32)
            + bl_ref[...])


def _tc_body_split(a0_ref, a1_ref, x0_ref, x1_ref, cnt_ref, wl_ref, bl_ref,
                   wr_ref, o0_ref, o1_ref):
    res = _tc_compute(a0_ref, a1_ref, x0_ref, x1_ref, cnt_ref, wl_ref,
                      bl_ref, wr_ref)
    o0_ref[...] = res[:, :H]
    o1_ref[...] = res[:, H:]


def _tc_body_final(a0_ref, a1_ref, x0_ref, x1_ref, cnt_ref, wl_ref, bl_ref,
                   wr_ref, o_ref):
    o_ref[...] = _tc_compute(a0_ref, a1_ref, x0_ref, x1_ref, cnt_ref,
                             wl_ref, bl_ref, wr_ref)


def _tc_layer(a0, a1, x0, x1, cnt, Wl, bl, Wr, final):
    half = pl.BlockSpec((BM, H), lambda i: (i, 0))
    in_specs = [
        half, half, half, half,
        pl.BlockSpec((BM, 1), lambda i: (i, 0)),
        pl.BlockSpec((D, D), lambda i: (0, 0)),
        pl.BlockSpec((1, D), lambda i: (0, 0)),
        pl.BlockSpec((D, D), lambda i: (0, 0)),
    ]
    if final:
        out_shape = jax.ShapeDtypeStruct((N, D), jnp.float32)
        out_spec = pl.BlockSpec((BM, D), lambda i: (i, 0))
        body = _tc_body_final
    else:
        out_shape = [jax.ShapeDtypeStruct((N, H), jnp.float32)] * 2
        out_spec = [half, half]
        body = _tc_body_split
    return pl.pallas_call(
        body,
        grid=(N // BM,),
        in_specs=in_specs,
        out_specs=out_spec,
        out_shape=out_shape,
    )(a0, a1, x0, x1, cnt, Wl, bl.reshape(1, D), Wr)


def kernel(x, edge_index, Wl0, bl0, Wr0, Wl1, bl1, Wr1):
    ei3 = edge_index.astype(jnp.int32).reshape(2, E // CHUNK, CHUNK)
    x0 = x[:, :H]
    x1 = x[:, H:]

    a0, a1, cnt = _sc_segsum_cnt(x0, x1, ei3)
    cnt2 = cnt[:N].reshape(N, 1)
    h0, h1 = _tc_layer(a0, a1, x0, x1, cnt2, Wl0, bl0, Wr0, final=False)
    b0, b1 = _sc_segsum_nocnt(h0, h1, ei3)
    return _tc_layer(b0, b1, h0, h1, cnt2, Wl1, bl1, Wr1, final=True)
